# baseline (speedup 1.0000x reference)
import jax
import jax.numpy as jnp
from jax import lax
from jax.experimental import pallas as pl
from jax.experimental.pallas import tpu as pltpu
from jax.experimental.pallas import tpu_sc as plsc

TOK = 8192
D_IN = 2048
D_OUT = 2048
E = 8
K = 2
NOISE_STD = 0.1

M = 256
NASSIGN = TOK * K
NPAD = NASSIGN + E * M
NBLK = NPAD // M

NC = 2
NS = 16
NW = NC * NS
RW = NPAD // NW
TW = TOK // NW
GC = 8
GNB = 6
GK = 4
CC = 8

_LANES = 128


def _gating_body(x_ref, gw_ref, nz_ref, gb_ref, pv_ref, iv_ref, loss_ref,
                 acc_ref):
    i = pl.program_id(0)
    logits = jnp.dot(x_ref[...], gw_ref[...],
                     preferred_element_type=jnp.float32)
    logits = logits + gb_ref[...] + nz_ref[...] * NOISE_STD
    lane = lax.broadcasted_iota(jnp.int32, logits.shape, 1)
    valid = lane < E
    logits = jnp.where(valid, logits, jnp.float32(-1e30))
    m = jnp.max(logits, axis=1, keepdims=True)
    e = jnp.where(valid, jnp.exp(logits - m), 0.0)
    probs = e / jnp.sum(e, axis=1, keepdims=True)

    @pl.when(i == 0)
    def _():
        acc_ref[...] = jnp.zeros_like(acc_ref)

    acc_ref[...] += jnp.sum(probs, axis=0, keepdims=True)

    v1 = jnp.max(probs, axis=1, keepdims=True)
    i1 = jnp.min(jnp.where(probs == v1, lane, 999), axis=1, keepdims=True)
    probs2 = jnp.where(lane == i1, -1.0, probs)
    v2 = jnp.max(probs2, axis=1, keepdims=True)
    i2 = jnp.min(jnp.where(probs2 == v2, lane, 999), axis=1, keepdims=True)
    pv_ref[...] = jnp.concatenate([v1, v2], axis=1)
    iv_ref[...] = jnp.concatenate([i1, i2], axis=1)

    @pl.when(i == pl.num_programs(0) - 1)
    def _():
        mean = acc_ref[...] * (1.0 / TOK)
        lane2 = lax.broadcasted_iota(jnp.int32, mean.shape, 1)
        dev = jnp.where(lane2 < E, mean - 1.0 / E, 0.0)
        loss_ref[...] = jnp.sum(dev * dev).reshape(1, 1)


def _gating(x, gwp, noise_p, gbp):
    bt = 1024
    grid = TOK // bt
    return pl.pallas_call(
        _gating_body,
        grid=(grid,),
        in_specs=[
            pl.BlockSpec((bt, D_IN), lambda i: (i, 0)),
            pl.BlockSpec((D_IN, _LANES), lambda i: (0, 0)),
            pl.BlockSpec((bt, _LANES), lambda i: (i, 0)),
            pl.BlockSpec((1, _LANES), lambda i: (0, 0)),
        ],
        out_specs=[
            pl.BlockSpec((bt, K), lambda i: (i, 0)),
            pl.BlockSpec((bt, K), lambda i: (i, 0)),
            pl.BlockSpec((1, 1), lambda i: (0, 0)),
        ],
        out_shape=[
            jax.ShapeDtypeStruct((TOK, K), jnp.float32),
            jax.ShapeDtypeStruct((TOK, K), jnp.int32),
            jax.ShapeDtypeStruct((1, 1), jnp.float32),
        ],
        scratch_shapes=[pltpu.VMEM((1, _LANES), jnp.float32)],
    )(x, gwp, noise_p, gbp)


def _gather_body(x_hbm, idx_hbm, out_hbm, idxall, *bufs_and_sems):
    rows = bufs_and_sems[:GNB]
    sg = bufs_and_sems[GNB:2 * GNB]
    sw = bufs_and_sems[2 * GNB:3 * GNB]
    wid = lax.axis_index("s") * NC + lax.axis_index("c")
    base = wid * RW
    pltpu.sync_copy(idx_hbm.at[pl.ds(base, RW)], idxall)
    nch = RW // GC

    for b in range(GK):
        pltpu.async_copy(x_hbm.at[idxall.at[pl.ds(b * GC, GC)]], rows[b],
                         sg[b])

    def grp(i, carry):
        g = i * GNB
        for b in range(GNB):
            ch = g + b
            pltpu.make_async_copy(x_hbm.at[pl.ds(0, GC), :], rows[b],
                                  sg[b]).wait()
            pltpu.async_copy(rows[b],
                             out_hbm.at[pl.ds(base + ch * GC, GC), :], sw[b])
            bn = (b + GK) % GNB

            @pl.when(ch >= GNB - GK)
            def _():
                pltpu.make_async_copy(rows[bn],
                                      out_hbm.at[pl.ds(base, GC), :],
                                      sw[bn]).wait()

            @pl.when(ch + GK < nch)
            def _():
                pltpu.async_copy(
                    x_hbm.at[idxall.at[pl.ds((ch + GK) * GC, GC)]],
                    rows[bn], sg[bn])
        return carry

    lax.fori_loop(0, nch // GNB, grp, 0)
    for ch in range(nch - (GNB - GK), nch):
        bl = ch % GNB
        pltpu.make_async_copy(rows[bl], out_hbm.at[pl.ds(base, GC), :],
                              sw[bl]).wait()


def _sc_gather(x, src_tok):
    k = pl.kernel(
        _gather_body,
        mesh=plsc.VectorSubcoreMesh(core_axis_name="c", subcore_axis_name="s"),
        out_type=jax.ShapeDtypeStruct((NPAD, D_IN), jnp.float32),
        scratch_types=(
            [pltpu.VMEM((RW,), jnp.int32)]
            + [pltpu.VMEM((GC, D_IN), jnp.float32)] * GNB
            + [pltpu.SemaphoreType.DMA] * (2 * GNB)
        ),
    )
    return k(x, src_tok)


def _gmm_body(be_ref, xg_ref, pg_ref, wt_ref, eb_ref, out_ref):
    xs = (xg_ref[...] * pg_ref[...]).astype(jnp.bfloat16)
    acc = jnp.dot(xs, wt_ref[0], preferred_element_type=jnp.float32)
    out_ref[...] = acc + pg_ref[...] * eb_ref[0]


def _gmm(block_expert, xg, pg, wt, eb):
    grid_spec = pltpu.PrefetchScalarGridSpec(
        num_scalar_prefetch=1,
        grid=(NBLK,),
        in_specs=[
            pl.BlockSpec((M, D_IN), lambda i, be: (i, 0)),
            pl.BlockSpec((M, 1), lambda i, be: (i, 0)),
            pl.BlockSpec((1, D_IN, D_OUT), lambda i, be: (be[i], 0, 0)),
            pl.BlockSpec((1, 1, D_OUT), lambda i, be: (be[i], 0, 0)),
        ],
        out_specs=pl.BlockSpec((M, D_OUT), lambda i, be: (i, 0)),
    )
    return pl.pallas_call(
        _gmm_body,
        grid_spec=grid_spec,
        out_shape=jax.ShapeDtypeStruct((NPAD, D_OUT), jnp.float32),
    )(block_expert, xg, pg, wt, eb)


def _combine_body(yg_hbm, d_hbm, out_hbm, dall, ab0, ab1, o0, o1,
                  sg0, sg1, sw0, sw1):
    wid = lax.axis_index("s") * NC + lax.axis_index("c")
    base = wid * TW
    pltpu.sync_copy(d_hbm.at[pl.ds(base * K, TW * K)], dall)
    AB = (ab0, ab1)
    O = (o0, o1)
    SG = (sg0, sg1)
    SW = (sw0, sw1)
    nch = TW // CC

    for p in range(2):
        pltpu.async_copy(yg_hbm.at[dall.at[pl.ds(p * K * CC, K * CC)]],
                         AB[p], SG[p])

    def pair(i, carry):
        g = i * 2
        for p in range(2):
            ch = g + p
            pltpu.make_async_copy(yg_hbm.at[pl.ds(0, K * CC), :], AB[p],
                                  SG[p]).wait()

            @pl.when(ch >= 2)
            def _():
                pltpu.make_async_copy(O[p], out_hbm.at[pl.ds(base, CC), :],
                                      SW[p]).wait()

            def row(r, c):
                def vec8(j, c2):
                    for u in range(8):
                        sl = pl.ds((j * 8 + u) * 16, 16)
                        O[p][r, sl] = AB[p][2 * r, sl] + AB[p][2 * r + 1, sl]
                    return c2
                return lax.fori_loop(0, D_OUT // 128, vec8, c)

            lax.fori_loop(0, CC, row, 0)

            @pl.when(ch + 2 < nch)
            def _():
                pltpu.async_copy(
                    yg_hbm.at[dall.at[pl.ds((ch + 2) * K * CC, K * CC)]],
                    AB[p], SG[p])

            pltpu.async_copy(O[p], out_hbm.at[pl.ds(base + ch * CC, CC), :],
                             SW[p])
        return carry

    lax.fori_loop(0, nch // 2, pair, 0)
    for p in range(2):
        pltpu.make_async_copy(O[p], out_hbm.at[pl.ds(base, CC), :],
                              SW[p]).wait()


def _sc_combine(yg, dest):
    k = pl.kernel(
        _combine_body,
        mesh=plsc.VectorSubcoreMesh(core_axis_name="c", subcore_axis_name="s"),
        out_type=jax.ShapeDtypeStruct((TOK, D_OUT), jnp.float32),
        scratch_types=[
            pltpu.VMEM((TW * K,), jnp.int32),
            pltpu.VMEM((K * CC, D_OUT), jnp.float32),
            pltpu.VMEM((K * CC, D_OUT), jnp.float32),
            pltpu.VMEM((CC, D_OUT), jnp.float32),
            pltpu.VMEM((CC, D_OUT), jnp.float32),
            pltpu.SemaphoreType.DMA,
            pltpu.SemaphoreType.DMA,
            pltpu.SemaphoreType.DMA,
            pltpu.SemaphoreType.DMA,
        ],
    )
    return k(yg, dest)


def kernel(x, gate_w, gate_b, experts_w, experts_b, noise):
    gwp = jnp.zeros((_LANES, D_IN), jnp.float32).at[:E].set(gate_w).T
    gbp = jnp.zeros((1, _LANES), jnp.float32).at[0, :E].set(gate_b)
    noise_p = jnp.zeros((TOK, _LANES), jnp.float32).at[:, :E].set(noise)
    wt = (experts_w.reshape(E, D_OUT, D_IN)
          .transpose(0, 2, 1).astype(jnp.bfloat16))
    eb = experts_b.reshape(E, 1, D_OUT)

    pv, iv, loss = _gating(x, gwp, noise_p, gbp)

    e_flat = iv.reshape(-1)
    p_flat = pv.reshape(-1)
    oh = (e_flat[:, None] == jnp.arange(E, dtype=jnp.int32)[None, :])
    csum = jnp.cumsum(oh.astype(jnp.int32), axis=0)
    rank = jnp.take_along_axis(csum, e_flat[:, None], axis=1)[:, 0] - 1
    counts = csum[-1]
    padded = ((counts + M - 1) // M) * M
    pcum = jnp.cumsum(padded)
    poff = jnp.concatenate([jnp.zeros((1,), pcum.dtype), pcum])[:E]
    dest = (poff[e_flat] + rank).astype(jnp.int32)
    arange_a = jnp.arange(NASSIGN, dtype=jnp.int32)
    src_tok = jnp.zeros((NPAD,), jnp.int32).at[dest].set(arange_a // K)
    pg = jnp.zeros((NPAD,), jnp.float32).at[dest].set(p_flat)
    bstart = jnp.arange(NBLK, dtype=jnp.int32) * M
    block_expert = jnp.minimum(
        jnp.sum((bstart[:, None] >= pcum[None, :]).astype(jnp.int32), axis=1),
        E - 1).astype(jnp.int32)

    xg = _sc_gather(x, src_tok)

    yg = _gmm(block_expert, xg, pg.reshape(NPAD, 1), wt, eb)

    out = _sc_combine(yg, dest)

    return (out, loss[0, 0])


# speedup vs baseline: 1.0634x; 1.0634x over previous
import jax
import jax.numpy as jnp
from jax import lax
from jax.experimental import pallas as pl
from jax.experimental.pallas import tpu as pltpu
from jax.experimental.pallas import tpu_sc as plsc

TOK = 8192
D_IN = 2048
D_OUT = 2048
E = 8
K = 2
NOISE_STD = 0.1

M = 256
NASSIGN = TOK * K
NPAD = NASSIGN + E * M
NBLK = NPAD // M

NC = 2
NS = 16
NW = NC * NS
RW = NPAD // NW
TW = TOK // NW
GC = 8
GNB = 6
GK = 4
CC = 8

_LANES = 128


def _gating_body(x_ref, gw_ref, nz_ref, gb_ref, pv_ref, iv_ref, loss_ref,
                 acc_ref):
    i = pl.program_id(0)
    logits = jnp.dot(x_ref[...], gw_ref[...],
                     preferred_element_type=jnp.float32)
    logits = logits + gb_ref[...] + nz_ref[...] * NOISE_STD
    lane = lax.broadcasted_iota(jnp.int32, logits.shape, 1)
    valid = lane < E
    logits = jnp.where(valid, logits, jnp.float32(-1e30))
    m = jnp.max(logits, axis=1, keepdims=True)
    e = jnp.where(valid, jnp.exp(logits - m), 0.0)
    probs = e / jnp.sum(e, axis=1, keepdims=True)

    @pl.when(i == 0)
    def _():
        acc_ref[...] = jnp.zeros_like(acc_ref)

    acc_ref[...] += jnp.sum(probs, axis=0, keepdims=True)

    v1 = jnp.max(probs, axis=1, keepdims=True)
    i1 = jnp.min(jnp.where(probs == v1, lane, 999), axis=1, keepdims=True)
    probs2 = jnp.where(lane == i1, -1.0, probs)
    v2 = jnp.max(probs2, axis=1, keepdims=True)
    i2 = jnp.min(jnp.where(probs2 == v2, lane, 999), axis=1, keepdims=True)
    pv_ref[...] = jnp.concatenate([v1, v2], axis=1)
    iv_ref[...] = jnp.concatenate([i1, i2], axis=1)

    @pl.when(i == pl.num_programs(0) - 1)
    def _():
        mean = acc_ref[...] * (1.0 / TOK)
        lane2 = lax.broadcasted_iota(jnp.int32, mean.shape, 1)
        dev = jnp.where(lane2 < E, mean - 1.0 / E, 0.0)
        loss_ref[...] = jnp.sum(dev * dev).reshape(1, 1)


def _gating(x, gwp, noise_p, gbp):
    bt = 1024
    grid = TOK // bt
    return pl.pallas_call(
        _gating_body,
        grid=(grid,),
        in_specs=[
            pl.BlockSpec((bt, D_IN), lambda i: (i, 0)),
            pl.BlockSpec((D_IN, _LANES), lambda i: (0, 0)),
            pl.BlockSpec((bt, _LANES), lambda i: (i, 0)),
            pl.BlockSpec((1, _LANES), lambda i: (0, 0)),
        ],
        out_specs=[
            pl.BlockSpec((bt, K), lambda i: (i, 0)),
            pl.BlockSpec((bt, K), lambda i: (i, 0)),
            pl.BlockSpec((1, 1), lambda i: (0, 0)),
        ],
        out_shape=[
            jax.ShapeDtypeStruct((TOK, K), jnp.float32),
            jax.ShapeDtypeStruct((TOK, K), jnp.int32),
            jax.ShapeDtypeStruct((1, 1), jnp.float32),
        ],
        scratch_shapes=[pltpu.VMEM((1, _LANES), jnp.float32)],
    )(x, gwp, noise_p, gbp)


def _gather_body(x_hbm, idx_hbm, out_hbm, idxall, *bufs_and_sems):
    rows = bufs_and_sems[:GNB]
    sg = bufs_and_sems[GNB:2 * GNB]
    sw = bufs_and_sems[2 * GNB:3 * GNB]
    wid = lax.axis_index("s") * NC + lax.axis_index("c")
    base = wid * RW
    pltpu.sync_copy(idx_hbm.at[pl.ds(base, RW)], idxall)
    nch = RW // GC

    for b in range(GK):
        pltpu.async_copy(x_hbm.at[idxall.at[pl.ds(b * GC, GC)]], rows[b],
                         sg[b])

    def grp(i, carry):
        g = i * GNB
        for b in range(GNB):
            ch = g + b
            pltpu.make_async_copy(x_hbm.at[pl.ds(0, GC), :], rows[b],
                                  sg[b]).wait()
            pltpu.async_copy(rows[b],
                             out_hbm.at[pl.ds(base + ch * GC, GC), :], sw[b])
            bn = (b + GK) % GNB

            @pl.when(ch >= GNB - GK)
            def _():
                pltpu.make_async_copy(rows[bn],
                                      out_hbm.at[pl.ds(base, GC), :],
                                      sw[bn]).wait()

            @pl.when(ch + GK < nch)
            def _():
                pltpu.async_copy(
                    x_hbm.at[idxall.at[pl.ds((ch + GK) * GC, GC)]],
                    rows[bn], sg[bn])
        return carry

    lax.fori_loop(0, nch // GNB, grp, 0)
    for ch in range(nch - (GNB - GK), nch):
        bl = ch % GNB
        pltpu.make_async_copy(rows[bl], out_hbm.at[pl.ds(base, GC), :],
                              sw[bl]).wait()


def _sc_gather(x, src_tok):
    k = pl.kernel(
        _gather_body,
        mesh=plsc.VectorSubcoreMesh(core_axis_name="c", subcore_axis_name="s"),
        out_type=jax.ShapeDtypeStruct((NPAD, D_IN), jnp.float32),
        scratch_types=(
            [pltpu.VMEM((RW,), jnp.int32)]
            + [pltpu.VMEM((GC, D_IN), jnp.float32)] * GNB
            + [pltpu.SemaphoreType.DMA] * (2 * GNB)
        ),
    )
    return k(x, src_tok)


def _gmm_body(be_ref, xg_ref, pg_ref, wt_ref, eb_ref, out_ref):
    xs = (xg_ref[...] * pg_ref[...]).astype(jnp.bfloat16)
    acc = jnp.dot(xs, wt_ref[0], preferred_element_type=jnp.float32)
    out_ref[...] = acc + pg_ref[...] * eb_ref[0]


def _gmm(block_expert, xg, pg, wt, eb):
    grid_spec = pltpu.PrefetchScalarGridSpec(
        num_scalar_prefetch=1,
        grid=(NBLK,),
        in_specs=[
            pl.BlockSpec((M, D_IN), lambda i, be: (i, 0)),
            pl.BlockSpec((M, 1), lambda i, be: (i, 0)),
            pl.BlockSpec((1, D_IN, D_OUT), lambda i, be: (be[i], 0, 0)),
            pl.BlockSpec((1, 1, D_OUT), lambda i, be: (be[i], 0, 0)),
        ],
        out_specs=pl.BlockSpec((M, D_OUT), lambda i, be: (i, 0)),
    )
    return pl.pallas_call(
        _gmm_body,
        grid_spec=grid_spec,
        out_shape=jax.ShapeDtypeStruct((NPAD, D_OUT), jnp.float32),
    )(block_expert, xg, pg, wt, eb)


def _combine_body(yg_hbm, d0_hbm, d1_hbm, out_hbm, d0all, d1all,
                  *bufs_and_sems):
    O = bufs_and_sems[0:4]
    B = bufs_and_sems[4:6]
    SA = bufs_and_sems[6:10]
    SB = bufs_and_sems[10:12]
    SW = bufs_and_sems[12:16]
    wid = lax.axis_index("s") * NC + lax.axis_index("c")
    base = wid * TW
    pltpu.sync_copy(d0_hbm.at[pl.ds(base, TW)], d0all)
    pltpu.sync_copy(d1_hbm.at[pl.ds(base, TW)], d1all)
    nch = TW // CC

    for p in range(2):
        pltpu.async_copy(yg_hbm.at[d0all.at[pl.ds(p * CC, CC)]], O[p], SA[p])
        pltpu.async_copy(yg_hbm.at[d1all.at[pl.ds(p * CC, CC)]], B[p], SB[p])

    def grp(i, carry):
        g = i * 4
        for b in range(4):
            ch = g + b
            pb = b % 2
            pltpu.make_async_copy(yg_hbm.at[pl.ds(0, CC), :], O[b],
                                  SA[b]).wait()
            pltpu.make_async_copy(yg_hbm.at[pl.ds(0, CC), :], B[pb],
                                  SB[pb]).wait()

            def row(r, c):
                def vec8(j, c2):
                    for u in range(8):
                        sl = pl.ds((j * 8 + u) * 16, 16)
                        plsc.addupdate(O[b].at[r, sl], B[pb][r, sl])
                    return c2
                return lax.fori_loop(0, D_OUT // 128, vec8, c)

            lax.fori_loop(0, CC, row, 0)
            pltpu.async_copy(O[b], out_hbm.at[pl.ds(base + ch * CC, CC), :],
                             SW[b])
            bn = (b + 2) % 4

            @pl.when(ch + 2 < nch)
            def _():
                @pl.when(ch >= 2)
                def _():
                    pltpu.make_async_copy(O[bn],
                                          out_hbm.at[pl.ds(base, CC), :],
                                          SW[bn]).wait()
                pltpu.async_copy(
                    yg_hbm.at[d0all.at[pl.ds((ch + 2) * CC, CC)]],
                    O[bn], SA[bn])
                pltpu.async_copy(
                    yg_hbm.at[d1all.at[pl.ds((ch + 2) * CC, CC)]],
                    B[pb], SB[pb])
        return carry

    lax.fori_loop(0, nch // 4, grp, 0)
    for b in range(4):
        pltpu.make_async_copy(O[b], out_hbm.at[pl.ds(base, CC), :],
                              SW[b]).wait()


def _sc_combine(yg, d0, d1):
    k = pl.kernel(
        _combine_body,
        mesh=plsc.VectorSubcoreMesh(core_axis_name="c", subcore_axis_name="s"),
        out_type=jax.ShapeDtypeStruct((TOK, D_OUT), jnp.float32),
        scratch_types=(
            [pltpu.VMEM((TW,), jnp.int32)] * 2
            + [pltpu.VMEM((CC, D_OUT), jnp.float32)] * 6
            + [pltpu.SemaphoreType.DMA] * 10
        ),
    )
    return k(yg, d0, d1)


def kernel(x, gate_w, gate_b, experts_w, experts_b, noise):
    gwp = jnp.zeros((_LANES, D_IN), jnp.float32).at[:E].set(gate_w).T
    gbp = jnp.zeros((1, _LANES), jnp.float32).at[0, :E].set(gate_b)
    noise_p = jnp.zeros((TOK, _LANES), jnp.float32).at[:, :E].set(noise)
    wt = (experts_w.reshape(E, D_OUT, D_IN)
          .transpose(0, 2, 1).astype(jnp.bfloat16))
    eb = experts_b.reshape(E, 1, D_OUT)

    pv, iv, loss = _gating(x, gwp, noise_p, gbp)

    e_flat = iv.reshape(-1)
    p_flat = pv.reshape(-1)
    oh = (e_flat[:, None] == jnp.arange(E, dtype=jnp.int32)[None, :])
    csum = jnp.cumsum(oh.astype(jnp.int32), axis=0)
    rank = jnp.take_along_axis(csum, e_flat[:, None], axis=1)[:, 0] - 1
    counts = csum[-1]
    padded = ((counts + M - 1) // M) * M
    pcum = jnp.cumsum(padded)
    poff = jnp.concatenate([jnp.zeros((1,), pcum.dtype), pcum])[:E]
    dest = (poff[e_flat] + rank).astype(jnp.int32)
    arange_a = jnp.arange(NASSIGN, dtype=jnp.int32)
    src_tok = jnp.zeros((NPAD,), jnp.int32).at[dest].set(arange_a // K)
    pg = jnp.zeros((NPAD,), jnp.float32).at[dest].set(p_flat)
    bstart = jnp.arange(NBLK, dtype=jnp.int32) * M
    block_expert = jnp.minimum(
        jnp.sum((bstart[:, None] >= pcum[None, :]).astype(jnp.int32), axis=1),
        E - 1).astype(jnp.int32)

    xg = _sc_gather(x, src_tok)

    yg = _gmm(block_expert, xg, pg.reshape(NPAD, 1), wt, eb)

    out = _sc_combine(yg, dest[0::K], dest[1::K])

    return (out, loss[0, 0])


# speedup vs baseline: 1.4012x; 1.3176x over previous
import jax
import jax.numpy as jnp
from jax import lax
from jax.experimental import pallas as pl
from jax.experimental.pallas import tpu as pltpu
from jax.experimental.pallas import tpu_sc as plsc

TOK = 8192
D_IN = 2048
D_OUT = 2048
E = 8
K = 2
NOISE_STD = 0.1

M = 256
NASSIGN = TOK * K
NPAD = NASSIGN + E * M
NBLK = NPAD // M

NC = 2
NS = 16
NW = NC * NS
RW = NPAD // NW
TW = TOK // NW
GC2 = 8
DNB = 4
CC = 8

_LANES = 128


def _gating_body(x_ref, gw_ref, nz_ref, gb_ref, pv_ref, iv_ref, loss_ref,
                 acc_ref):
    i = pl.program_id(0)
    logits = jnp.dot(x_ref[...], gw_ref[...],
                     preferred_element_type=jnp.float32)
    logits = logits + gb_ref[...] + nz_ref[...] * NOISE_STD
    lane = lax.broadcasted_iota(jnp.int32, logits.shape, 1)
    valid = lane < E
    logits = jnp.where(valid, logits, jnp.float32(-1e30))
    m = jnp.max(logits, axis=1, keepdims=True)
    e = jnp.where(valid, jnp.exp(logits - m), 0.0)
    probs = e / jnp.sum(e, axis=1, keepdims=True)

    @pl.when(i == 0)
    def _():
        acc_ref[...] = jnp.zeros_like(acc_ref)

    acc_ref[...] += jnp.sum(probs, axis=0, keepdims=True)

    v1 = jnp.max(probs, axis=1, keepdims=True)
    i1 = jnp.min(jnp.where(probs == v1, lane, 999), axis=1, keepdims=True)
    probs2 = jnp.where(lane == i1, -1.0, probs)
    v2 = jnp.max(probs2, axis=1, keepdims=True)
    i2 = jnp.min(jnp.where(probs2 == v2, lane, 999), axis=1, keepdims=True)
    pv_ref[...] = jnp.concatenate([v1, v2], axis=1)
    iv_ref[...] = jnp.concatenate([i1, i2], axis=1)

    @pl.when(i == pl.num_programs(0) - 1)
    def _():
        mean = acc_ref[...] * (1.0 / TOK)
        lane2 = lax.broadcasted_iota(jnp.int32, mean.shape, 1)
        dev = jnp.where(lane2 < E, mean - 1.0 / E, 0.0)
        loss_ref[...] = jnp.sum(dev * dev).reshape(1, 1)


def _gating(x, gwp, noise_p, gbp):
    bt = 1024
    grid = TOK // bt
    return pl.pallas_call(
        _gating_body,
        grid=(grid,),
        in_specs=[
            pl.BlockSpec((bt, D_IN), lambda i: (i, 0)),
            pl.BlockSpec((D_IN, _LANES), lambda i: (0, 0)),
            pl.BlockSpec((bt, _LANES), lambda i: (i, 0)),
            pl.BlockSpec((1, _LANES), lambda i: (0, 0)),
        ],
        out_specs=[
            pl.BlockSpec((bt, K), lambda i: (i, 0)),
            pl.BlockSpec((bt, K), lambda i: (i, 0)),
            pl.BlockSpec((1, 1), lambda i: (0, 0)),
        ],
        out_shape=[
            jax.ShapeDtypeStruct((TOK, K), jnp.float32),
            jax.ShapeDtypeStruct((TOK, K), jnp.int32),
            jax.ShapeDtypeStruct((1, 1), jnp.float32),
        ],
        scratch_shapes=[pltpu.VMEM((1, _LANES), jnp.float32)],
    )(x, gwp, noise_p, gbp)


def _dist_body(x_hbm, d0_hbm, d1_hbm, out_hbm, d0v, d1v, *bufs_and_sems):
    S = bufs_and_sems[0:DNB]
    SF = bufs_and_sems[DNB:2 * DNB]
    S0 = bufs_and_sems[2 * DNB:3 * DNB]
    S1 = bufs_and_sems[3 * DNB:4 * DNB]
    wid = lax.axis_index("s") * NC + lax.axis_index("c")
    tb = wid * TW
    pltpu.sync_copy(d0_hbm.at[wid], d0v)
    pltpu.sync_copy(d1_hbm.at[wid], d1v)
    nch = TW // GC2

    for b in range(2):
        pltpu.async_copy(x_hbm.at[pl.ds(tb + b * GC2, GC2), :], S[b], SF[b])

    def grp(i, carry):
        g = i * DNB
        for b in range(DNB):
            ch = g + b
            pltpu.make_async_copy(x_hbm.at[pl.ds(0, GC2), :], S[b],
                                  SF[b]).wait()
            pltpu.async_copy(S[b], out_hbm.at[d0v.at[ch]], S0[b])
            pltpu.async_copy(S[b], out_hbm.at[d1v.at[ch]], S1[b])
            bn = (b + 2) % DNB

            @pl.when(ch >= 2)
            def _():
                pltpu.make_async_copy(S[bn], out_hbm.at[pl.ds(0, GC2), :],
                                      S0[bn]).wait()
                pltpu.make_async_copy(S[bn], out_hbm.at[pl.ds(0, GC2), :],
                                      S1[bn]).wait()

            @pl.when(ch + 2 < nch)
            def _():
                pltpu.async_copy(x_hbm.at[pl.ds(tb + (ch + 2) * GC2, GC2), :],
                                 S[bn], SF[bn])
        return carry

    lax.fori_loop(0, nch // DNB, grp, 0)
    for ch in range(nch - 2, nch):
        bl = ch % DNB
        pltpu.make_async_copy(S[bl], out_hbm.at[pl.ds(0, GC2), :],
                              S0[bl]).wait()
        pltpu.make_async_copy(S[bl], out_hbm.at[pl.ds(0, GC2), :],
                              S1[bl]).wait()


def _sc_distribute(x, d0tm, d1tm):
    k = pl.kernel(
        _dist_body,
        mesh=plsc.VectorSubcoreMesh(core_axis_name="c", subcore_axis_name="s"),
        out_type=jax.ShapeDtypeStruct((NPAD, D_IN), jnp.float32),
        scratch_types=(
            [pltpu.VMEM((TW // GC2, GC2), jnp.int32)] * 2
            + [pltpu.VMEM((GC2, D_IN), jnp.float32)] * DNB
            + [pltpu.SemaphoreType.DMA] * (3 * DNB)
        ),
    )
    return k(x, d0tm, d1tm)


def _gmm_body(be_ref, xg_ref, pg_ref, wt_ref, eb_ref, out_ref):
    pgv = pg_ref[...]
    xs = jnp.where(pgv > 0, xg_ref[...] * pgv, 0.0).astype(jnp.bfloat16)
    acc = jnp.dot(xs, wt_ref[0], preferred_element_type=jnp.float32)
    out_ref[...] = acc + pg_ref[...] * eb_ref[0]


def _gmm(block_expert, xg, pg, wt, eb):
    grid_spec = pltpu.PrefetchScalarGridSpec(
        num_scalar_prefetch=1,
        grid=(NBLK,),
        in_specs=[
            pl.BlockSpec((M, D_IN), lambda i, be: (i, 0)),
            pl.BlockSpec((M, 1), lambda i, be: (i, 0)),
            pl.BlockSpec((1, D_IN, D_OUT), lambda i, be: (be[i], 0, 0)),
            pl.BlockSpec((1, 1, D_OUT), lambda i, be: (be[i], 0, 0)),
        ],
        out_specs=pl.BlockSpec((M, D_OUT), lambda i, be: (i, 0)),
    )
    return pl.pallas_call(
        _gmm_body,
        grid_spec=grid_spec,
        out_shape=jax.ShapeDtypeStruct((NPAD, D_OUT), jnp.float32),
    )(block_expert, xg, pg, wt, eb)


def _combine_body(yg_hbm, d0_hbm, d1_hbm, out_hbm, d0all, d1all,
                  *bufs_and_sems):
    O = bufs_and_sems[0:4]
    B = bufs_and_sems[4:6]
    SA = bufs_and_sems[6:10]
    SB = bufs_and_sems[10:12]
    SW = bufs_and_sems[12:16]
    wid = lax.axis_index("s") * NC + lax.axis_index("c")
    base = wid * TW
    pltpu.sync_copy(d0_hbm.at[pl.ds(base, TW)], d0all)
    pltpu.sync_copy(d1_hbm.at[pl.ds(base, TW)], d1all)
    nch = TW // CC

    for p in range(2):
        pltpu.async_copy(yg_hbm.at[d0all.at[pl.ds(p * CC, CC)]], O[p], SA[p])
        pltpu.async_copy(yg_hbm.at[d1all.at[pl.ds(p * CC, CC)]], B[p], SB[p])

    def grp(i, carry):
        g = i * 4
        for b in range(4):
            ch = g + b
            pb = b % 2
            pltpu.make_async_copy(yg_hbm.at[pl.ds(0, CC), :], O[b],
                                  SA[b]).wait()
            pltpu.make_async_copy(yg_hbm.at[pl.ds(0, CC), :], B[pb],
                                  SB[pb]).wait()

            def row(r, c):
                def vec8(j, c2):
                    for u in range(8):
                        sl = pl.ds((j * 8 + u) * 16, 16)
                        plsc.addupdate(O[b].at[r, sl], B[pb][r, sl])
                    return c2
                return lax.fori_loop(0, D_OUT // 128, vec8, c)

            lax.fori_loop(0, CC, row, 0)
            pltpu.async_copy(O[b], out_hbm.at[pl.ds(base + ch * CC, CC), :],
                             SW[b])
            bn = (b + 2) % 4

            @pl.when(ch + 2 < nch)
            def _():
                @pl.when(ch >= 2)
                def _():
                    pltpu.make_async_copy(O[bn],
                                          out_hbm.at[pl.ds(base, CC), :],
                                          SW[bn]).wait()
                pltpu.async_copy(
                    yg_hbm.at[d0all.at[pl.ds((ch + 2) * CC, CC)]],
                    O[bn], SA[bn])
                pltpu.async_copy(
                    yg_hbm.at[d1all.at[pl.ds((ch + 2) * CC, CC)]],
                    B[pb], SB[pb])
        return carry

    lax.fori_loop(0, nch // 4, grp, 0)
    for b in range(4):
        pltpu.make_async_copy(O[b], out_hbm.at[pl.ds(base, CC), :],
                              SW[b]).wait()


def _sc_combine(yg, d0, d1):
    k = pl.kernel(
        _combine_body,
        mesh=plsc.VectorSubcoreMesh(core_axis_name="c", subcore_axis_name="s"),
        out_type=jax.ShapeDtypeStruct((TOK, D_OUT), jnp.float32),
        scratch_types=(
            [pltpu.VMEM((TW,), jnp.int32)] * 2
            + [pltpu.VMEM((CC, D_OUT), jnp.float32)] * 6
            + [pltpu.SemaphoreType.DMA] * 10
        ),
    )
    return k(yg, d0, d1)


def kernel(x, gate_w, gate_b, experts_w, experts_b, noise):
    gwp = jnp.zeros((_LANES, D_IN), jnp.float32).at[:E].set(gate_w).T
    gbp = jnp.zeros((1, _LANES), jnp.float32).at[0, :E].set(gate_b)
    noise_p = jnp.zeros((TOK, _LANES), jnp.float32).at[:, :E].set(noise)
    wt = (experts_w.reshape(E, D_OUT, D_IN)
          .transpose(0, 2, 1).astype(jnp.bfloat16))
    eb = experts_b.reshape(E, 1, D_OUT)

    pv, iv, loss = _gating(x, gwp, noise_p, gbp)

    e_flat = iv.reshape(-1)
    p_flat = pv.reshape(-1)
    oh = (e_flat[:, None] == jnp.arange(E, dtype=jnp.int32)[None, :])
    csum = jnp.cumsum(oh.astype(jnp.int32), axis=0)
    rank = jnp.take_along_axis(csum, e_flat[:, None], axis=1)[:, 0] - 1
    counts = csum[-1]
    padded = ((counts + M - 1) // M) * M
    pcum = jnp.cumsum(padded)
    poff = jnp.concatenate([jnp.zeros((1,), pcum.dtype), pcum])[:E]
    dest = (poff[e_flat] + rank).astype(jnp.int32)
    pg = jnp.zeros((NPAD,), jnp.float32).at[dest].set(p_flat)
    bstart = jnp.arange(NBLK, dtype=jnp.int32) * M
    block_expert = jnp.minimum(
        jnp.sum((bstart[:, None] >= pcum[None, :]).astype(jnp.int32), axis=1),
        E - 1).astype(jnp.int32)

    d0 = dest[0::K]
    d1 = dest[1::K]
    xg = _sc_distribute(x, d0.reshape(NW, TW // GC2, GC2),
                        d1.reshape(NW, TW // GC2, GC2))

    yg = _gmm(block_expert, xg, pg.reshape(NPAD, 1), wt, eb)

    out = _sc_combine(yg, d0, d1)

    return (out, loss[0, 0])
